# baseline (speedup 1.0000x reference)
import functools

import jax
import jax.numpy as jnp
from jax import lax
from jax.experimental import pallas as pl
from jax.experimental.pallas import tpu as pltpu
from jax.experimental.pallas import tpu_sc as plsc

HID = 768
HALF = HID // 2
EPS = 1e-12
NC, NS, L = 2, 16, 16
NW = NC * NS
GCH = 32
BLK = 1024


def _pack_row(rows, bf, row, brow):
    for kk in range(HALF // L):
        ks = pl.ds(kk * L, L)
        a = plsc.bitcast(rows[row, ks], jnp.int32)
        b = plsc.bitcast(rows[row, pl.ds(HALF + kk * L, L)], jnp.int32)
        lo = ((a + 0x8000) >> 16) & 0xFFFF
        hi = (b + 0x8000) & jnp.int32(-65536)
        bf[brow, ks] = lo | hi


def _gather_body(ids_hbm, word_hbm, out_hbm, idx_v, rows, bf, sem_g0, sem_g1,
                 sem_s0, sem_s1, sem_i):
    tok = ids_hbm.shape[0]
    tpw = tok // NW
    n_ch = tpw // GCH

    wid = lax.axis_index("s") * NC + lax.axis_index("c")
    base = wid * tpw

    pltpu.async_copy(ids_hbm.at[pl.ds(base, tpw)], idx_v, sem_i).wait()

    gsems = [sem_g0, sem_g1]
    ssems = [sem_s0, sem_s1]
    copies = [None, None]
    scats = [None, None]

    def pack_chunk(slot):
        def body(r, _):
            _pack_row(rows, bf, slot * GCH + r, slot * GCH + r)
            return 0
        lax.fori_loop(0, GCH, body, 0)

    for g in range(n_ch):
        slot = g % 2
        if scats[slot] is not None:
            scats[slot].wait()
        copies[slot] = pltpu.async_copy(
            word_hbm.at[idx_v.at[pl.ds(g * GCH, GCH)]],
            rows.at[pl.ds(slot * GCH, GCH)], gsems[slot])
        if g > 0:
            pslot = (g - 1) % 2
            copies[pslot].wait()
            pack_chunk(pslot)
            scats[pslot] = pltpu.async_copy(
                bf.at[pl.ds(pslot * GCH, GCH)],
                out_hbm.at[pl.ds(base + (g - 1) * GCH, GCH)], ssems[pslot])
    last = n_ch - 1
    lslot = last % 2
    copies[lslot].wait()
    pack_chunk(lslot)
    scats[lslot] = pltpu.async_copy(
        bf.at[pl.ds(lslot * GCH, GCH)],
        out_hbm.at[pl.ds(base + last * GCH, GCH)], ssems[lslot])
    for sl in range(2):
        if scats[sl] is not None:
            scats[sl].wait()


@jax.jit
def _sc_gather(ids, word_emb):
    tok = ids.shape[0]
    mesh = plsc.VectorSubcoreMesh(core_axis_name="c", subcore_axis_name="s")
    k = pl.kernel(
        _gather_body,
        out_type=jax.ShapeDtypeStruct((tok, HALF), jnp.int32),
        mesh=mesh,
        compiler_params=pltpu.CompilerParams(needs_layout_passes=False),
        scratch_types=[
            pltpu.VMEM((tok // NW,), jnp.int32),
            pltpu.VMEM((2 * GCH, HID), jnp.float32),
            pltpu.VMEM((2 * GCH, HALF), jnp.int32),
            pltpu.SemaphoreType.DMA,
            pltpu.SemaphoreType.DMA,
            pltpu.SemaphoreType.DMA,
            pltpu.SemaphoreType.DMA,
            pltpu.SemaphoreType.DMA,
        ],
    )
    return k(ids, word_emb)


def _ln_body(words_ref, pos_ref, tt_ref, at_ref, type_ref, tag_ref,
             gam_ref, bet_ref, out_ref):
    u = words_ref[...]
    wa = lax.bitcast_convert_type(u << 16, jnp.float32)
    wb = lax.bitcast_convert_type(u & jnp.int32(-65536), jnp.float32)
    tt = tt_ref[0]
    at = at_ref[0]

    def half(w, lo):
        hs = slice(lo, lo + HALF)
        t_rows = jnp.where(tt == 0, type_ref[0, hs][None, :],
                           type_ref[1, hs][None, :])
        a_rows = jnp.where(at == 0, tag_ref[0, hs][None, :],
                           jnp.where(at == 1, tag_ref[1, hs][None, :],
                                     tag_ref[2, hs][None, :]))
        return w + pos_ref[:, hs] + t_rows + a_rows

    ea = half(wa, 0)
    eb = half(wb, HALF)
    s1 = jnp.sum(ea, axis=1, keepdims=True) + jnp.sum(eb, axis=1, keepdims=True)
    s2 = (jnp.sum(ea * ea, axis=1, keepdims=True)
          + jnp.sum(eb * eb, axis=1, keepdims=True))
    mean = s1 * (1.0 / HID)
    var = s2 * (1.0 / HID) - mean * mean
    r = lax.rsqrt(var + EPS)
    out_ref[:, :HALF] = ((ea - mean) * r * gam_ref[0, :HALF][None, :]
                         + bet_ref[0, :HALF][None, :])
    out_ref[:, HALF:] = ((eb - mean) * r * gam_ref[0, HALF:][None, :]
                         + bet_ref[0, HALF:][None, :])


@jax.jit
def _tc_ln(words, pos_emb, tt3, at3, type_emb, tag_emb, gamma, beta):
    tok = words.shape[0]
    seq = pos_emb.shape[0]
    n_s = seq // BLK
    n_b = tok // seq
    grid = (n_s, n_b)

    return pl.pallas_call(
        _ln_body,
        grid=grid,
        in_specs=[
            pl.BlockSpec((BLK, HALF), lambda j, b: (b * n_s + j, 0)),
            pl.BlockSpec((BLK, HID), lambda j, b: (j, 0)),
            pl.BlockSpec((1, BLK, 1), lambda j, b: (b * n_s + j, 0, 0)),
            pl.BlockSpec((1, BLK, 1), lambda j, b: (b * n_s + j, 0, 0)),
            pl.BlockSpec((2, HID), lambda j, b: (0, 0)),
            pl.BlockSpec((3, HID), lambda j, b: (0, 0)),
            pl.BlockSpec((1, HID), lambda j, b: (0, 0)),
            pl.BlockSpec((1, HID), lambda j, b: (0, 0)),
        ],
        out_specs=pl.BlockSpec((BLK, HID), lambda j, b: (b * n_s + j, 0)),
        out_shape=jax.ShapeDtypeStruct((tok, HID), jnp.float32),
    )(words, pos_emb, tt3, at3, type_emb, tag_emb, gamma, beta)


def kernel(input_ids, token_type_ids, answer_tag_ids, word_emb, pos_emb,
           type_emb, tag_emb, ln_gamma, ln_beta):
    b, s = input_ids.shape
    tok = b * s
    ids = input_ids.reshape(-1).astype(jnp.int32)
    tt3 = token_type_ids.astype(jnp.int32).reshape(tok // BLK, BLK, 1)
    at3 = answer_tag_ids.astype(jnp.int32).reshape(tok // BLK, BLK, 1)
    words = _sc_gather(ids, word_emb)
    out = _tc_ln(words, pos_emb, tt3, at3, type_emb, tag_emb,
                 ln_gamma.reshape(1, HID), ln_beta.reshape(1, HID))
    return out.reshape(b, s, HID)
